# baseline (speedup 1.0000x reference)
import jax
import jax.numpy as jnp
from jax.experimental import pallas as pl
from jax.experimental.pallas import tpu as pltpu

N_BLK = 1000


def _mlp(layers, x):
    n = len(layers)
    for i, (W, b) in enumerate(layers):
        x = x @ W + b
        if i < n - 1:
            x = jax.nn.relu(x)
    return x


def _lstm(seq, layers):
    xs = jnp.transpose(seq, (1, 0, 2))
    h_last = None
    for (Wih, Whh, bih, bhh) in layers:
        Hh = Whh.shape[1]
        Nb = xs.shape[1]
        def step(carry, xt, Wih=Wih, Whh=Whh, bih=bih, bhh=bhh):
            h, c = carry
            g = xt @ Wih.T + h @ Whh.T + bih + bhh
            i, f, gg, o = jnp.split(g, 4, axis=-1)
            c = jax.nn.sigmoid(f) * c + jax.nn.sigmoid(i) * jnp.tanh(gg)
            h = jax.nn.sigmoid(o) * jnp.tanh(c)
            return (h, c), h
        init = (jnp.zeros((Nb, Hh), dtype=xs.dtype), jnp.zeros((Nb, Hh), dtype=xs.dtype))
        (h, c), ys = jax.lax.scan(step, init, xs)
        xs = ys
        h_last = h
    return h_last


def _dec_kernel(h_ref, xt_ref, dt_ref, w1_ref, b1_ref, w2_ref, b2_ref, o_ref):
    a = jnp.maximum(h_ref[...] @ w1_ref[...] + b1_ref[...], 0.0)
    d = a @ w2_ref[...] + b2_ref[...]
    o_ref[...] = xt_ref[...] + d * dt_ref[...]


def kernel(x, node_mass, pos, edge_attr, delta_t, params, edge_index, edge_surf_index):
    n = x.shape[0]
    x_t = x[:, :, -1]
    seq = jnp.transpose(x, (0, 2, 1))
    h = _lstm(seq, params['lstm'])
    h = jnp.concatenate([h, node_mass[:, None], pos], axis=-1)
    h_topo = _mlp(params['temp_fc'], h)
    mat_id = edge_attr[:, 0].astype(jnp.int32)
    emb = jnp.take(params['mat_emb'], mat_id, axis=0)
    edge_feat = _mlp(params['edge_mlp'], jnp.concatenate([emb, edge_attr[:, 1:]], axis=-1))
    ssrc = edge_surf_index[0]
    sdst = edge_surf_index[1]
    rel = pos[ssrc] - pos[sdst]
    nrm = jnp.sqrt(jnp.sum(rel * rel, axis=-1, keepdims=True) + 1e-12)
    ef = _mlp(params['surf_edge'], jnp.concatenate([rel, nrm], axis=-1))
    agg_s = jax.ops.segment_sum(ef, sdst, num_segments=n)
    h_surf = _mlp(params['surf_node'], agg_s)
    mask = jnp.zeros((n,), dtype=h_surf.dtype).at[edge_surf_index.reshape(-1)].set(1.0)
    h_final = h_topo + h_surf * mask[:, None]
    src = edge_index[0]
    dst = edge_index[1]
    for blk in params['gnn']:
        e_in = jnp.concatenate([edge_feat, h_final[src], h_final[dst]], axis=-1)
        edge_feat = edge_feat + _mlp(blk['edge'], e_in)
        agg = jax.ops.segment_sum(edge_feat, dst, num_segments=n)
        h_final = h_final + _mlp(blk['node'], jnp.concatenate([h_final, agg], axis=-1))

    (w1, b1), (w2, b2) = params['dec']
    grid = n // N_BLK
    out = pl.pallas_call(
        _dec_kernel,
        grid=(grid,),
        in_specs=[
            pl.BlockSpec((N_BLK, 64), lambda i: (i, 0)),
            pl.BlockSpec((N_BLK, 12), lambda i: (i, 0)),
            pl.BlockSpec((N_BLK, 1), lambda i: (i, 0)),
            pl.BlockSpec((64, 64), lambda i: (0, 0)),
            pl.BlockSpec((64,), lambda i: (0,)),
            pl.BlockSpec((64, 12), lambda i: (0, 0)),
            pl.BlockSpec((12,), lambda i: (0,)),
        ],
        out_specs=pl.BlockSpec((N_BLK, 12), lambda i: (i, 0)),
        out_shape=jax.ShapeDtypeStruct((n, 12), jnp.float32),
    )(h_final, x_t, delta_t[:, None], w1, b1, w2, b2)
    return out


# speedup vs baseline: 1.1896x; 1.1896x over previous
import functools

import jax
import jax.numpy as jnp
from jax.experimental import pallas as pl
from jax.experimental.pallas import tpu as pltpu

N_BLK = 1000
E_BLK = 4000


def _node_enc_kernel(x0, x1, x2, x3, mp,
                     wx1, wh1, b1, wx2, wh2, b2,
                     wfh, wfmp, bf1, wf2, bf2, o_ref):
    xs = (x0[...], x1[...], x2[...], x3[...])
    B = xs[0].shape[0]
    dt = jnp.float32

    def lstm_layer(inputs, wx, wh, b):
        h = jnp.zeros((B, 64), dt)
        c = jnp.zeros((B, 64), dt)
        outs = []
        for xt in inputs:
            g = xt @ wx[...] + h @ wh[...] + b[...]
            i, f, gg, o = jnp.split(g, 4, axis=-1)
            c = jax.nn.sigmoid(f) * c + jax.nn.sigmoid(i) * jnp.tanh(gg)
            h = jax.nn.sigmoid(o) * jnp.tanh(c)
            outs.append(h)
        return outs

    h1 = lstm_layer(xs, wx1, wh1, b1)
    h2 = lstm_layer(h1, wx2, wh2, b2)
    h = h2[-1]
    a = jnp.maximum(h @ wfh[...] + mp[...] @ wfmp[...] + bf1[...], 0.0)
    o_ref[...] = a @ wf2[...] + bf2[...]


def _edge_enc_kernel(mid, attr, wemb, wattr, b1, w2, b2, o_ref):
    B = mid.shape[0]
    oh = (mid[...] == jax.lax.broadcasted_iota(jnp.int32, (B, 8), 1)).astype(jnp.float32)
    a = jnp.maximum(oh @ wemb[...] + attr[...] @ wattr[...] + b1[...], 0.0)
    o_ref[...] = a @ w2[...] + b2[...]


def _surf_edge_kernel(psrc, pdst, wrel, wnrm, b1, w2, b2, o_ref):
    rel = psrc[...] - pdst[...]
    nrm = jnp.sqrt(jnp.sum(rel * rel, axis=-1, keepdims=True) + 1e-12)
    a = jnp.maximum(rel @ wrel[...] + nrm @ wnrm[...] + b1[...], 0.0)
    o_ref[...] = a @ w2[...] + b2[...]


def _surf_node_kernel(htopo, agg, cnt, w1, b1, w2, b2, o_ref):
    a = jnp.maximum(agg[...] @ w1[...] + b1[...], 0.0)
    hs = a @ w2[...] + b2[...]
    mask = (cnt[...] > 0.0).astype(jnp.float32)
    o_ref[...] = htopo[...] + hs * mask


def _gnn_edge_kernel(ef, hs, hd, w0e, w0s, w0d, b0, w2, b2, o_ref):
    a = jnp.maximum(ef[...] @ w0e[...] + hs[...] @ w0s[...] + hd[...] @ w0d[...]
                    + b0[...], 0.0)
    o_ref[...] = ef[...] + a @ w2[...] + b2[...]


def _gnn_node_kernel(h, agg, w1h, w1a, b1, w2, b2, o_ref):
    a = jnp.maximum(h[...] @ w1h[...] + agg[...] @ w1a[...] + b1[...], 0.0)
    o_ref[...] = h[...] + a @ w2[...] + b2[...]


def _dec_kernel(h, xt, dt, w1, b1, w2, b2, o_ref):
    a = jnp.maximum(h[...] @ w1[...] + b1[...], 0.0)
    d = a @ w2[...] + b2[...]
    o_ref[...] = xt[...] + d * dt[...]


def _row_spec(b, d):
    return pl.BlockSpec((b, d), lambda i: (i, 0))


def _full_spec(*shape):
    nd = len(shape)
    return pl.BlockSpec(shape, lambda i: (0,) * nd)


def _tc_call(kern, grid, row_args, full_args, out_rows, out_cols, interpret=False):
    in_specs = ([_row_spec(a[1], a[0].shape[-1]) for a in row_args]
                + [_full_spec(*a.shape) for a in full_args])
    return pl.pallas_call(
        kern,
        grid=(grid,),
        in_specs=in_specs,
        out_specs=_row_spec(out_rows, out_cols),
        out_shape=jax.ShapeDtypeStruct((grid * out_rows, out_cols), jnp.float32),
        interpret=interpret,
    )(*[a[0] for a in row_args], *full_args)


def kernel(x, node_mass, pos, edge_attr, delta_t, params, edge_index,
           edge_surf_index, interpret=False):
    n = x.shape[0]
    E = edge_index.shape[1]
    ES = edge_surf_index.shape[1]
    f32 = jnp.float32

    (Wih1, Whh1, bih1, bhh1), (Wih2, Whh2, bih2, bhh2) = params['lstm']
    wx1, wh1, b1 = Wih1.T, Whh1.T, bih1 + bhh1
    wx2, wh2, b2 = Wih2.T, Whh2.T, bih2 + bhh2
    (Wf1, bf1), (Wf2, bf2) = params['temp_fc']
    wfh, wfmp = Wf1[:64], Wf1[64:]
    (We1, be1), (We2, be2) = params['edge_mlp']
    wemb = params['mat_emb'] @ We1[:4]
    wattr = We1[4:]
    (Ws1, bs1), (Ws2, bs2) = params['surf_edge']
    wrel, wnrm = Ws1[:3], Ws1[3:]
    (Wn1, bn1), (Wn2, bn2) = params['surf_node']
    (Wd1, bd1), (Wd2, bd2) = params['dec']

    xts = [x[:, :, t] for t in range(4)]
    mp = jnp.concatenate([node_mass[:, None], pos], axis=-1)
    grid_n = n // N_BLK
    h_topo = pl.pallas_call(
        _node_enc_kernel,
        grid=(grid_n,),
        in_specs=[_row_spec(N_BLK, 12)] * 4 + [_row_spec(N_BLK, 4)]
        + [_full_spec(*w.shape) for w in
           (wx1, wh1, b1, wx2, wh2, b2, wfh, wfmp, bf1, Wf2, bf2)],
        out_specs=_row_spec(N_BLK, 64),
        out_shape=jax.ShapeDtypeStruct((n, 64), f32),
        interpret=interpret,
    )(*xts, mp, wx1, wh1, b1, wx2, wh2, b2, wfh, wfmp, bf1, Wf2, bf2)

    mat_id = edge_attr[:, :1].astype(jnp.int32)
    attr = edge_attr[:, 1:]
    edge_feat = _tc_call(
        _edge_enc_kernel, E // E_BLK,
        [(mat_id, E_BLK), (attr, E_BLK)],
        [wemb, wattr, be1, We2, be2], E_BLK, 64, interpret)

    ssrc = edge_surf_index[0]
    sdst = edge_surf_index[1]
    psrc = pos[ssrc]
    pdst = pos[sdst]
    ef_s = _tc_call(
        _surf_edge_kernel, ES // E_BLK,
        [(psrc, E_BLK), (pdst, E_BLK)],
        [wrel, wnrm, bs1, Ws2, bs2], E_BLK, 64, interpret)
    agg_s = jax.ops.segment_sum(ef_s, sdst, num_segments=n)
    cnt = jnp.zeros((n,), f32).at[edge_surf_index.reshape(-1)].add(1.0)
    h_final = _tc_call(
        _surf_node_kernel, grid_n,
        [(h_topo, N_BLK), (agg_s, N_BLK), (cnt[:, None], N_BLK)],
        [Wn1, bn1, Wn2, bn2], N_BLK, 64, interpret)

    src = edge_index[0]
    dst = edge_index[1]
    for blk in params['gnn']:
        (Wg1, bg1), (Wg2, bg2) = blk['edge']
        (Wb1, bb1), (Wb2, bb2) = blk['node']
        hs = h_final[src]
        hd = h_final[dst]
        edge_feat = _tc_call(
            _gnn_edge_kernel, E // E_BLK,
            [(edge_feat, E_BLK), (hs, E_BLK), (hd, E_BLK)],
            [Wg1[:64], Wg1[64:128], Wg1[128:], bg1, Wg2, bg2],
            E_BLK, 64, interpret)
        agg = jax.ops.segment_sum(edge_feat, dst, num_segments=n)
        h_final = _tc_call(
            _gnn_node_kernel, grid_n,
            [(h_final, N_BLK), (agg, N_BLK)],
            [Wb1[:64], Wb1[64:], bb1, Wb2, bb2], N_BLK, 64, interpret)

    out = _tc_call(
        _dec_kernel, grid_n,
        [(h_final, N_BLK), (x[:, :, -1], N_BLK), (delta_t[:, None], N_BLK)],
        [Wd1, bd1, Wd2, bd2], N_BLK, 12, interpret)
    return out


# speedup vs baseline: 1.5992x; 1.3444x over previous
import functools

import jax
import jax.numpy as jnp
from jax.experimental import pallas as pl
from jax.experimental.pallas import tpu as pltpu
from jax.experimental.pallas import tpu_sc as plsc

N_BLK = 1000
E_BLK = 4096

_NW = 32
_CHUNK = 128
_GRP = 5
_NOUT = 10000
_NACC = 10008


def _sc_mesh():
    return plsc.VectorSubcoreMesh(core_axis_name="c", subcore_axis_name="s",
                                  num_cores=2, num_subcores=16)


def _sc_gather(table, idx3d):
    D = table.shape[1]
    nw, nchunks, ck = idx3d.shape
    grp = _GRP
    ngrp = nchunks // grp
    rows_w = nchunks * ck
    M = nw * rows_w

    @functools.partial(
        pl.kernel,
        out_type=jax.ShapeDtypeStruct((M, D), jnp.float32),
        mesh=_sc_mesh(),
        scratch_types=[
            pltpu.VMEM((nchunks, ck), jnp.int32),
            pltpu.VMEM((grp * ck, D), jnp.float32),
            pltpu.SemaphoreType.DMA,
        ],
    )
    def k(table_hbm, idx_hbm, out_hbm, idx_v, rows_v, sem):
        wid = jax.lax.axis_index("s") * 2 + jax.lax.axis_index("c")
        pltpu.sync_copy(idx_hbm.at[wid], idx_v)

        def body(g, _):
            descs = [
                pltpu.async_copy(table_hbm.at[idx_v.at[g * grp + i]],
                                 rows_v.at[pl.ds(i * ck, ck)], sem)
                for i in range(grp)
            ]
            for d in descs:
                d.wait()
            pltpu.sync_copy(rows_v,
                            out_hbm.at[pl.ds(wid * rows_w + g * grp * ck, grp * ck)])
            return 0

        jax.lax.fori_loop(0, ngrp, body, 0)

    return k(table, idx3d)


def _zero_acc(zrow_hbm, acc_sh, sid, ck):
    full = _NACC // ck
    rem = _NACC - full * ck
    for j in range(5):
        slot = sid * 5 + j

        @pl.when(slot < full)
        def _():
            pltpu.sync_copy(zrow_hbm, acc_sh.at[pl.ds(slot * ck, ck)])

        @pl.when(slot == full)
        def _():
            pltpu.sync_copy(zrow_hbm.at[pl.ds(0, rem)],
                            acc_sh.at[pl.ds(slot * ck, rem)])


def _sc_scatter_add(vals, idx3d, zrow):
    D = vals.shape[1]
    nw, nchunks, ck = idx3d.shape
    grp = 2
    ngrp = nchunks // grp
    rows_w = nchunks * ck

    @functools.partial(
        pl.kernel,
        out_type=jax.ShapeDtypeStruct((2, _NOUT, D), jnp.float32),
        mesh=_sc_mesh(),
        scratch_types=[
            pltpu.VMEM((nchunks, ck), jnp.int32),
            pltpu.VMEM((grp * ck, D), jnp.float32),
            pltpu.VMEM_SHARED((_NACC, D), jnp.float32),
            pltpu.SemaphoreType.DMA,
        ],
    )
    def k(vals_hbm, idx_hbm, zrow_hbm, out_hbm, idx_v, rows_v, acc_sh, sem):
        cid = jax.lax.axis_index("c")
        sid = jax.lax.axis_index("s")
        wid = sid * 2 + cid

        _zero_acc(zrow_hbm, acc_sh, sid, ck)
        pltpu.sync_copy(idx_hbm.at[wid], idx_v)
        plsc.subcore_barrier()

        def body(g, _):
            pltpu.sync_copy(
                vals_hbm.at[pl.ds(wid * rows_w + g * grp * ck, grp * ck)], rows_v)
            for i in range(grp):
                pltpu.sync_copy(rows_v.at[pl.ds(i * ck, ck)],
                                acc_sh.at[idx_v.at[g * grp + i]], add=True)
            return 0

        jax.lax.fori_loop(0, ngrp, body, 0)
        plsc.subcore_barrier()

        @pl.when(sid == 0)
        def _():
            pltpu.sync_copy(acc_sh.at[pl.ds(0, _NOUT)], out_hbm.at[cid])

    return k(vals, idx3d, zrow)


def _sc_scatter_surf(vals, dst3d, src3d, zrow, ones_hbm):
    D = vals.shape[1]
    nw, nchunks, ck = dst3d.shape
    rows_w = nchunks * ck

    @functools.partial(
        pl.kernel,
        out_type=jax.ShapeDtypeStruct((2, _NOUT, D), jnp.float32),
        mesh=_sc_mesh(),
        scratch_types=[
            pltpu.VMEM((nchunks, ck), jnp.int32),
            pltpu.VMEM((nchunks, ck), jnp.int32),
            pltpu.VMEM((ck, D), jnp.float32),
            pltpu.VMEM((ck, D), jnp.float32),
            pltpu.VMEM_SHARED((_NACC, D), jnp.float32),
            pltpu.SemaphoreType.DMA,
        ],
    )
    def k(vals_hbm, dst_hbm, src_hbm, zrow_hbm, ones_h,
          out_hbm, idxd_v, idxs_v, rows_v, ones_v, acc_sh, sem):
        cid = jax.lax.axis_index("c")
        sid = jax.lax.axis_index("s")
        wid = sid * 2 + cid

        _zero_acc(zrow_hbm, acc_sh, sid, ck)
        pltpu.sync_copy(dst_hbm.at[wid], idxd_v)
        pltpu.sync_copy(src_hbm.at[wid], idxs_v)
        pltpu.sync_copy(ones_h, ones_v)
        plsc.subcore_barrier()
        for i in range(nchunks):
            pltpu.sync_copy(vals_hbm.at[pl.ds(wid * rows_w + i * ck, ck)], rows_v)
            pltpu.sync_copy(rows_v, acc_sh.at[idxd_v.at[i]], add=True)
            pltpu.sync_copy(ones_v, acc_sh.at[idxs_v.at[i]], add=True)
        plsc.subcore_barrier()

        @pl.when(sid == 0)
        def _():
            pltpu.sync_copy(acc_sh.at[pl.ds(0, _NOUT)], out_hbm.at[cid])

    return k(vals, dst3d, src3d, zrow, ones_hbm)


def _node_enc_kernel(x0, x1, x2, x3, mp,
                     wx1, wh1, b1, wx2, wh2, b2,
                     wfh, wfmp, bf1, wf2, bf2, o_ref):
    xs = (x0[...], x1[...], x2[...], x3[...])
    B = xs[0].shape[0]
    dt = jnp.float32

    def lstm_layer(inputs, wx, wh, b):
        h = jnp.zeros((B, 64), dt)
        c = jnp.zeros((B, 64), dt)
        outs = []
        for xt in inputs:
            g = xt @ wx[...] + h @ wh[...] + b[...]
            i, f, gg, o = jnp.split(g, 4, axis=-1)
            c = jax.nn.sigmoid(f) * c + jax.nn.sigmoid(i) * jnp.tanh(gg)
            h = jax.nn.sigmoid(o) * jnp.tanh(c)
            outs.append(h)
        return outs

    h1 = lstm_layer(xs, wx1, wh1, b1)
    h2 = lstm_layer(h1, wx2, wh2, b2)
    h = h2[-1]
    a = jnp.maximum(h @ wfh[...] + mp[...] @ wfmp[...] + bf1[...], 0.0)
    o_ref[...] = a @ wf2[...] + bf2[...]


def _edge_enc_kernel(mid, attr, wemb, wattr, b1, w2, b2, o_ref):
    B = mid.shape[0]
    oh = (mid[...] == jax.lax.broadcasted_iota(jnp.int32, (B, 8), 1)).astype(jnp.float32)
    a = jnp.maximum(oh @ wemb[...] + attr[...] @ wattr[...] + b1[...], 0.0)
    o_ref[...] = a @ w2[...] + b2[...]


def _surf_edge_kernel(psrc, pdst, wrel, wnrm, b1, w2, b2, o_ref):
    rel = psrc[...] - pdst[...]
    nrm = jnp.sqrt(jnp.sum(rel * rel, axis=-1, keepdims=True) + 1e-12)
    a = jnp.maximum(rel @ wrel[...] + nrm @ wnrm[...] + b1[...], 0.0)
    o_ref[...] = a @ w2[...] + b2[...]


def _surf_node_kernel(htopo, aggp, e64, w1, b1, w2, b2, o_ref):
    u = aggp[0] + aggp[1]
    a = jnp.maximum(u @ w1[...] + b1[...], 0.0)
    hs = a @ w2[...] + b2[...]
    mask = (u @ e64[...] > 0.0).astype(jnp.float32)
    o_ref[...] = htopo[...] + hs * mask


def _gnn_edge_kernel(ef, hs, hd, w0e, w0s, w0d, b0, w2, b2, o_ref):
    a = jnp.maximum(ef[...] @ w0e[...] + hs[...] @ w0s[...] + hd[...] @ w0d[...]
                    + b0[...], 0.0)
    o_ref[...] = ef[...] + a @ w2[...] + b2[...]


def _gnn_node_kernel(h, aggp, w1h, w1a, b1, w2, b2, o_ref):
    agg = aggp[0] + aggp[1]
    a = jnp.maximum(h[...] @ w1h[...] + agg @ w1a[...] + b1[...], 0.0)
    o_ref[...] = h[...] + a @ w2[...] + b2[...]


def _dec_kernel(h, xt, dt, w1, b1, w2, b2, o_ref):
    a = jnp.maximum(h[...] @ w1[...] + b1[...], 0.0)
    d = a @ w2[...] + b2[...]
    o_ref[...] = xt[...] + d * dt[...]


def _row_spec(b, *rest):
    nd = 1 + len(rest)
    if nd == 2:
        return pl.BlockSpec((b, rest[0]), lambda i: (i, 0))
    return pl.BlockSpec((rest[0], b, rest[1]), lambda i: (0, i, 0))


def _full_spec(*shape):
    nd = len(shape)
    return pl.BlockSpec(shape, lambda i: (0,) * nd)


def _tc_call(kern, grid, row_args, full_args, out_rows, out_cols,
             interpret=False):
    in_specs = []
    for a, b in row_args:
        if a.ndim == 2:
            in_specs.append(_row_spec(b, a.shape[-1]))
        else:
            in_specs.append(_row_spec(b, a.shape[0], a.shape[-1]))
    in_specs += [_full_spec(*a.shape) for a in full_args]
    return pl.pallas_call(
        kern,
        grid=(grid,),
        in_specs=in_specs,
        out_specs=_row_spec(out_rows, out_cols),
        out_shape=jax.ShapeDtypeStruct((grid * out_rows, out_cols), jnp.float32),
        interpret=interpret,
    )(*[a for a, _ in row_args], *full_args)


def _pad_rows(w, rows=128):
    return jnp.pad(w, ((0, rows - w.shape[0]), (0, 0)))


def _pad_cols(w, cols=128):
    return jnp.pad(w, ((0, 0), (0, cols - w.shape[1])))


def _pad_vec(b, cols=128, one_at=None):
    v = jnp.pad(b, (0, cols - b.shape[0]))
    if one_at is not None:
        v = v.at[one_at].set(1.0)
    return v


def kernel(x, node_mass, pos, edge_attr, delta_t, params, edge_index,
           edge_surf_index, interpret=False):
    n = x.shape[0]
    E = edge_index.shape[1]
    ES = edge_surf_index.shape[1]
    f32 = jnp.float32
    EPAD = ((E + _NW * _CHUNK - 1) // (_NW * _CHUNK)) * _NW * _CHUNK
    ESPAD = ((ES + _NW * _CHUNK - 1) // (_NW * _CHUNK)) * _NW * _CHUNK

    (Wih1, Whh1, bih1, bhh1), (Wih2, Whh2, bih2, bhh2) = params['lstm']
    wx1, wh1, b1 = Wih1.T, Whh1.T, bih1 + bhh1
    wx2, wh2, b2 = Wih2.T, Whh2.T, bih2 + bhh2
    (Wf1, bf1), (Wf2, bf2) = params['temp_fc']
    wfh, wfmp = Wf1[:64], Wf1[64:]
    (We1, be1), (We2, be2) = params['edge_mlp']
    wemb = params['mat_emb'] @ We1[:4]
    wattr = We1[4:]
    (Ws1, bs1), (Ws2, bs2) = params['surf_edge']
    wrel, wnrm = Ws1[:3], Ws1[3:]
    (Wn1, bn1), (Wn2, bn2) = params['surf_node']
    (Wd1, bd1), (Wd2, bd2) = params['dec']
    e64 = jnp.zeros((128, 1), f32).at[64, 0].set(1.0)

    src = edge_index[0].astype(jnp.int32)
    dst = edge_index[1].astype(jnp.int32)
    ssrc = edge_surf_index[0].astype(jnp.int32)
    sdst = edge_surf_index[1].astype(jnp.int32)
    gchunks = 2 * EPAD // (_NW * _CHUNK)
    schunks = EPAD // (_NW * _CHUNK)
    sgchunks = 2 * ESPAD // (_NW * _CHUNK)
    sschunks = ESPAD // (_NW * _CHUNK)
    gidx = jnp.concatenate([jnp.pad(src, (0, EPAD - E)),
                            jnp.pad(dst, (0, EPAD - E))]
                           ).reshape(_NW, gchunks, _CHUNK)
    dst3d = jnp.pad(dst, (0, EPAD - E), constant_values=_NOUT
                    ).reshape(_NW, schunks, _CHUNK)
    pidx = jnp.concatenate([jnp.pad(ssrc, (0, ESPAD - ES)),
                            jnp.pad(sdst, (0, ESPAD - ES))]
                           ).reshape(_NW, sgchunks, _CHUNK)
    sdst3d = jnp.pad(sdst, (0, ESPAD - ES), constant_values=_NOUT
                     ).reshape(_NW, sschunks, _CHUNK)
    ssrc3d = jnp.pad(ssrc, (0, ESPAD - ES), constant_values=_NOUT
                     ).reshape(_NW, sschunks, _CHUNK)
    zrow = jnp.zeros((_CHUNK, 128), f32)
    ones128 = jnp.zeros((_CHUNK, 128), f32).at[:, 64].set(1.0)
    pospad = jnp.pad(pos, ((0, 0), (0, 125)))

    xts = [x[:, :, t] for t in range(4)]
    mp = jnp.concatenate([node_mass[:, None], pos], axis=-1)
    grid_n = n // N_BLK
    wf2e, bf2e = _pad_cols(Wf2), _pad_vec(bf2)
    h_topo = pl.pallas_call(
        _node_enc_kernel,
        grid=(grid_n,),
        in_specs=[_row_spec(N_BLK, 12)] * 4 + [_row_spec(N_BLK, 4)]
        + [_full_spec(*w.shape) for w in
           (wx1, wh1, b1, wx2, wh2, b2, wfh, wfmp, bf1, wf2e, bf2e)],
        out_specs=_row_spec(N_BLK, 128),
        out_shape=jax.ShapeDtypeStruct((n, 128), f32),
        interpret=interpret,
    )(*xts, mp, wx1, wh1, b1, wx2, wh2, b2, wfh, wfmp, bf1, wf2e, bf2e)

    ea_pad = jnp.pad(edge_attr, ((0, EPAD - E), (0, 0)))
    mat_id = ea_pad[:, :1].astype(jnp.int32)
    attr = ea_pad[:, 1:]
    edge_feat = _tc_call(
        _edge_enc_kernel, EPAD // E_BLK,
        [(mat_id, E_BLK), (attr, E_BLK)],
        [wemb, wattr, be1, _pad_cols(We2), _pad_vec(be2, one_at=64)],
        E_BLK, 128, interpret)

    pg = _sc_gather(pospad, pidx)
    ef_s = _tc_call(
        _surf_edge_kernel, ESPAD // E_BLK,
        [(pg[:ESPAD], E_BLK), (pg[ESPAD:], E_BLK)],
        [_pad_rows(wrel), wnrm, bs1, _pad_cols(Ws2), _pad_vec(bs2, one_at=64)],
        E_BLK, 128, interpret)
    aggp = _sc_scatter_surf(ef_s, sdst3d, ssrc3d, zrow, ones128)
    h_final = _tc_call(
        _surf_node_kernel, grid_n,
        [(h_topo, N_BLK), (aggp, N_BLK)],
        [e64, _pad_rows(Wn1), bn1, _pad_cols(Wn2), _pad_vec(bn2)],
        N_BLK, 128, interpret)

    for blk in params['gnn']:
        (Wg1, bg1), (Wg2, bg2) = blk['edge']
        (Wb1, bb1), (Wb2, bb2) = blk['node']
        hg = _sc_gather(h_final, gidx)
        edge_feat = _tc_call(
            _gnn_edge_kernel, EPAD // E_BLK,
            [(edge_feat, E_BLK), (hg[:EPAD], E_BLK), (hg[EPAD:], E_BLK)],
            [_pad_rows(Wg1[:64]), _pad_rows(Wg1[64:128]), _pad_rows(Wg1[128:]),
             bg1, _pad_cols(Wg2), _pad_vec(bg2)],
            E_BLK, 128, interpret)
        aggp = _sc_scatter_add(edge_feat, dst3d, zrow)
        h_final = _tc_call(
            _gnn_node_kernel, grid_n,
            [(h_final, N_BLK), (aggp, N_BLK)],
            [_pad_rows(Wb1[:64]), _pad_rows(Wb1[64:]), bb1,
             _pad_cols(Wb2), _pad_vec(bb2)],
            N_BLK, 128, interpret)

    out = _tc_call(
        _dec_kernel, grid_n,
        [(h_final, N_BLK), (x[:, :, -1], N_BLK), (delta_t[:, None], N_BLK)],
        [_pad_rows(Wd1), bd1, Wd2, bd2], N_BLK, 12, interpret)
    return out


# speedup vs baseline: 1.6239x; 1.0154x over previous
import functools

import jax
import jax.numpy as jnp
from jax.experimental import pallas as pl
from jax.experimental.pallas import tpu as pltpu
from jax.experimental.pallas import tpu_sc as plsc

N_BLK = 1000
E_BLK = 4096

_NW = 32
_CHUNK = 128
_GRP = 5
_NOUT = 10000
_NACC = 10008


def _sc_mesh():
    return plsc.VectorSubcoreMesh(core_axis_name="c", subcore_axis_name="s",
                                  num_cores=2, num_subcores=16)


def _sc_gather(table, idx3d, nbuf=4):
    D = table.shape[1]
    nw, nchunks, ck = idx3d.shape
    ngrp = nchunks // nbuf
    rows_w = nchunks * ck
    M = nw * rows_w

    @functools.partial(
        pl.kernel,
        out_type=jax.ShapeDtypeStruct((M, D), jnp.float32),
        mesh=_sc_mesh(),
        scratch_types=[pltpu.VMEM((nchunks, ck), jnp.int32)]
        + [pltpu.VMEM((ck, D), jnp.float32)] * nbuf
        + [pltpu.SemaphoreType.DMA] * nbuf,
    )
    def k(table_hbm, idx_hbm, out_hbm, idx_v, *bufsem):
        bufs, sems = bufsem[:nbuf], bufsem[nbuf:]
        wid = jax.lax.axis_index("s") * 2 + jax.lax.axis_index("c")
        pltpu.sync_copy(idx_hbm.at[wid], idx_v)
        for b in range(nbuf):
            pltpu.async_copy(table_hbm.at[idx_v.at[b]], bufs[b], sems[b])

        def out_ref(c):
            return out_hbm.at[pl.ds(wid * rows_w + c * ck, ck)]

        def body(g, _):
            for b in range(nbuf):
                c = g * nbuf + b
                pltpu.make_async_copy(table_hbm.at[idx_v.at[c]],
                                      bufs[b], sems[b]).wait()
                pltpu.async_copy(bufs[b], out_ref(c), sems[b])
            for b in range(nbuf):
                c = g * nbuf + b

                @pl.when(c + nbuf < nchunks)
                def _():
                    pltpu.make_async_copy(bufs[b], out_ref(c), sems[b]).wait()
                    pltpu.async_copy(table_hbm.at[idx_v.at[c + nbuf]],
                                     bufs[b], sems[b])
            return 0

        jax.lax.fori_loop(0, ngrp, body, 0)
        for b in range(nbuf):
            pltpu.make_async_copy(bufs[b], out_ref(nchunks - nbuf + b),
                                  sems[b]).wait()

    return k(table, idx3d)


def _zero_acc(zrow_hbm, acc_sh, sid, ck):
    full = _NACC // ck
    rem = _NACC - full * ck
    for j in range(5):
        slot = sid * 5 + j

        @pl.when(slot < full)
        def _():
            pltpu.sync_copy(zrow_hbm, acc_sh.at[pl.ds(slot * ck, ck)])

        @pl.when(slot == full)
        def _():
            pltpu.sync_copy(zrow_hbm.at[pl.ds(0, rem)],
                            acc_sh.at[pl.ds(slot * ck, rem)])


def _sc_scatter_add(vals, idx3d, zrow, nbuf=2):
    D = vals.shape[1]
    nw, nchunks, ck = idx3d.shape
    ngrp = nchunks // nbuf
    rows_w = nchunks * ck

    @functools.partial(
        pl.kernel,
        out_type=jax.ShapeDtypeStruct((2, _NOUT, D), jnp.float32),
        mesh=_sc_mesh(),
        scratch_types=[
            pltpu.VMEM((nchunks, ck), jnp.int32),
            pltpu.VMEM_SHARED((_NACC, D), jnp.float32),
        ]
        + [pltpu.VMEM((ck, D), jnp.float32)] * nbuf
        + [pltpu.SemaphoreType.DMA] * nbuf,
    )
    def k(vals_hbm, idx_hbm, zrow_hbm, out_hbm, idx_v, acc_sh, *bufsem):
        bufs, sems = bufsem[:nbuf], bufsem[nbuf:]
        cid = jax.lax.axis_index("c")
        sid = jax.lax.axis_index("s")
        wid = sid * 2 + cid

        _zero_acc(zrow_hbm, acc_sh, sid, ck)
        pltpu.sync_copy(idx_hbm.at[wid], idx_v)
        plsc.subcore_barrier()

        def val_ref(c):
            return vals_hbm.at[pl.ds(wid * rows_w + c * ck, ck)]

        for b in range(nbuf):
            pltpu.async_copy(val_ref(b), bufs[b], sems[b])

        def body(g, _):
            for b in range(nbuf):
                c = g * nbuf + b
                pltpu.make_async_copy(val_ref(c), bufs[b], sems[b]).wait()
                pltpu.async_copy(bufs[b], acc_sh.at[idx_v.at[c]], sems[b],
                                 add=True)
            for b in range(nbuf):
                c = g * nbuf + b

                @pl.when(c + nbuf < nchunks)
                def _():
                    pltpu.make_async_copy(bufs[b], acc_sh.at[idx_v.at[c]],
                                          sems[b]).wait()
                    pltpu.async_copy(val_ref(c + nbuf), bufs[b], sems[b])
            return 0

        jax.lax.fori_loop(0, ngrp, body, 0)
        for b in range(nbuf):
            pltpu.make_async_copy(bufs[b],
                                  acc_sh.at[idx_v.at[nchunks - nbuf + b]],
                                  sems[b]).wait()
        plsc.subcore_barrier()

        @pl.when(sid == 0)
        def _():
            pltpu.sync_copy(acc_sh.at[pl.ds(0, _NOUT)], out_hbm.at[cid])

    return k(vals, idx3d, zrow)


def _sc_scatter_surf(vals, dst3d, src3d, zrow, ones_hbm):
    D = vals.shape[1]
    nw, nchunks, ck = dst3d.shape
    rows_w = nchunks * ck

    @functools.partial(
        pl.kernel,
        out_type=jax.ShapeDtypeStruct((2, _NOUT, D), jnp.float32),
        mesh=_sc_mesh(),
        scratch_types=[
            pltpu.VMEM((nchunks, ck), jnp.int32),
            pltpu.VMEM((nchunks, ck), jnp.int32),
            pltpu.VMEM((ck, D), jnp.float32),
            pltpu.VMEM((ck, D), jnp.float32),
            pltpu.VMEM_SHARED((_NACC, D), jnp.float32),
            pltpu.SemaphoreType.DMA,
        ],
    )
    def k(vals_hbm, dst_hbm, src_hbm, zrow_hbm, ones_h,
          out_hbm, idxd_v, idxs_v, rows_v, ones_v, acc_sh, sem):
        cid = jax.lax.axis_index("c")
        sid = jax.lax.axis_index("s")
        wid = sid * 2 + cid

        _zero_acc(zrow_hbm, acc_sh, sid, ck)
        pltpu.sync_copy(dst_hbm.at[wid], idxd_v)
        pltpu.sync_copy(src_hbm.at[wid], idxs_v)
        pltpu.sync_copy(ones_h, ones_v)
        plsc.subcore_barrier()
        for i in range(nchunks):
            pltpu.sync_copy(vals_hbm.at[pl.ds(wid * rows_w + i * ck, ck)], rows_v)
            pltpu.sync_copy(rows_v, acc_sh.at[idxd_v.at[i]], add=True)
            pltpu.sync_copy(ones_v, acc_sh.at[idxs_v.at[i]], add=True)
        plsc.subcore_barrier()

        @pl.when(sid == 0)
        def _():
            pltpu.sync_copy(acc_sh.at[pl.ds(0, _NOUT)], out_hbm.at[cid])

    return k(vals, dst3d, src3d, zrow, ones_hbm)


def _node_enc_kernel(x0, x1, x2, x3, mp,
                     wx1, wh1, b1, wx2, wh2, b2,
                     wfh, wfmp, bf1, wf2, bf2, o_ref):
    xs = (x0[...], x1[...], x2[...], x3[...])
    B = xs[0].shape[0]
    dt = jnp.float32

    def lstm_layer(inputs, wx, wh, b):
        h = jnp.zeros((B, 64), dt)
        c = jnp.zeros((B, 64), dt)
        outs = []
        for xt in inputs:
            g = xt @ wx[...] + h @ wh[...] + b[...]
            i, f, gg, o = jnp.split(g, 4, axis=-1)
            c = jax.nn.sigmoid(f) * c + jax.nn.sigmoid(i) * jnp.tanh(gg)
            h = jax.nn.sigmoid(o) * jnp.tanh(c)
            outs.append(h)
        return outs

    h1 = lstm_layer(xs, wx1, wh1, b1)
    h2 = lstm_layer(h1, wx2, wh2, b2)
    h = h2[-1]
    a = jnp.maximum(h @ wfh[...] + mp[...] @ wfmp[...] + bf1[...], 0.0)
    o_ref[...] = a @ wf2[...] + bf2[...]


def _edge_enc_kernel(mid, attr, wemb, wattr, b1, w2, b2, o_ref):
    B = mid.shape[0]
    oh = (mid[...] == jax.lax.broadcasted_iota(jnp.int32, (B, 8), 1)).astype(jnp.float32)
    a = jnp.maximum(oh @ wemb[...] + attr[...] @ wattr[...] + b1[...], 0.0)
    o_ref[...] = a @ w2[...] + b2[...]


def _surf_edge_kernel(psrc, pdst, wrel, wnrm, b1, w2, b2, o_ref):
    rel = psrc[...] - pdst[...]
    nrm = jnp.sqrt(jnp.sum(rel * rel, axis=-1, keepdims=True) + 1e-12)
    a = jnp.maximum(rel @ wrel[...] + nrm @ wnrm[...] + b1[...], 0.0)
    o_ref[...] = a @ w2[...] + b2[...]


def _surf_node_kernel(htopo, aggp, e64, w1, b1, w2, b2, o_ref):
    u = aggp[0] + aggp[1]
    a = jnp.maximum(u @ w1[...] + b1[...], 0.0)
    hs = a @ w2[...] + b2[...]
    mask = (u @ e64[...] > 0.0).astype(jnp.float32)
    o_ref[...] = htopo[...] + hs * mask


def _gnn_edge_kernel(ef, hs, hd, w0e, w0s, w0d, b0, w2, b2, o_ref):
    a = jnp.maximum(ef[...] @ w0e[...] + hs[...] @ w0s[...] + hd[...] @ w0d[...]
                    + b0[...], 0.0)
    o_ref[...] = ef[...] + a @ w2[...] + b2[...]


def _gnn_node_kernel(h, aggp, w1h, w1a, b1, w2, b2, o_ref):
    agg = aggp[0] + aggp[1]
    a = jnp.maximum(h[...] @ w1h[...] + agg @ w1a[...] + b1[...], 0.0)
    o_ref[...] = h[...] + a @ w2[...] + b2[...]


def _dec_kernel(h, xt, dt, w1, b1, w2, b2, o_ref):
    a = jnp.maximum(h[...] @ w1[...] + b1[...], 0.0)
    d = a @ w2[...] + b2[...]
    o_ref[...] = xt[...] + d * dt[...]


def _row_spec(b, *rest):
    nd = 1 + len(rest)
    if nd == 2:
        return pl.BlockSpec((b, rest[0]), lambda i: (i, 0))
    return pl.BlockSpec((rest[0], b, rest[1]), lambda i: (0, i, 0))


def _full_spec(*shape):
    nd = len(shape)
    return pl.BlockSpec(shape, lambda i: (0,) * nd)


def _tc_call(kern, grid, row_args, full_args, out_rows, out_cols,
             interpret=False):
    in_specs = []
    for a, b in row_args:
        if a.ndim == 2:
            in_specs.append(_row_spec(b, a.shape[-1]))
        else:
            in_specs.append(_row_spec(b, a.shape[0], a.shape[-1]))
    in_specs += [_full_spec(*a.shape) for a in full_args]
    return pl.pallas_call(
        kern,
        grid=(grid,),
        in_specs=in_specs,
        out_specs=_row_spec(out_rows, out_cols),
        out_shape=jax.ShapeDtypeStruct((grid * out_rows, out_cols), jnp.float32),
        interpret=interpret,
    )(*[a for a, _ in row_args], *full_args)


def _pad_rows(w, rows=128):
    return jnp.pad(w, ((0, rows - w.shape[0]), (0, 0)))


def _pad_cols(w, cols=128):
    return jnp.pad(w, ((0, 0), (0, cols - w.shape[1])))


def _pad_vec(b, cols=128, one_at=None):
    v = jnp.pad(b, (0, cols - b.shape[0]))
    if one_at is not None:
        v = v.at[one_at].set(1.0)
    return v


def kernel(x, node_mass, pos, edge_attr, delta_t, params, edge_index,
           edge_surf_index, interpret=False):
    n = x.shape[0]
    E = edge_index.shape[1]
    ES = edge_surf_index.shape[1]
    f32 = jnp.float32
    EPAD = ((E + _NW * _CHUNK - 1) // (_NW * _CHUNK)) * _NW * _CHUNK
    ESPAD = ((ES + _NW * _CHUNK - 1) // (_NW * _CHUNK)) * _NW * _CHUNK

    (Wih1, Whh1, bih1, bhh1), (Wih2, Whh2, bih2, bhh2) = params['lstm']
    wx1, wh1, b1 = Wih1.T, Whh1.T, bih1 + bhh1
    wx2, wh2, b2 = Wih2.T, Whh2.T, bih2 + bhh2
    (Wf1, bf1), (Wf2, bf2) = params['temp_fc']
    wfh, wfmp = Wf1[:64], Wf1[64:]
    (We1, be1), (We2, be2) = params['edge_mlp']
    wemb = params['mat_emb'] @ We1[:4]
    wattr = We1[4:]
    (Ws1, bs1), (Ws2, bs2) = params['surf_edge']
    wrel, wnrm = Ws1[:3], Ws1[3:]
    (Wn1, bn1), (Wn2, bn2) = params['surf_node']
    (Wd1, bd1), (Wd2, bd2) = params['dec']
    e64 = jnp.zeros((128, 1), f32).at[64, 0].set(1.0)

    src = edge_index[0].astype(jnp.int32)
    dst = edge_index[1].astype(jnp.int32)
    ssrc = edge_surf_index[0].astype(jnp.int32)
    sdst = edge_surf_index[1].astype(jnp.int32)
    gchunks = 2 * EPAD // (_NW * _CHUNK)
    schunks = EPAD // (_NW * _CHUNK)
    sgchunks = 2 * ESPAD // (_NW * _CHUNK)
    sschunks = ESPAD // (_NW * _CHUNK)
    gidx = jnp.concatenate([jnp.pad(src, (0, EPAD - E)),
                            jnp.pad(dst, (0, EPAD - E))]
                           ).reshape(_NW, gchunks, _CHUNK)
    dst3d = jnp.pad(dst, (0, EPAD - E), constant_values=_NOUT
                    ).reshape(_NW, schunks, _CHUNK)
    pidx = jnp.concatenate([jnp.pad(ssrc, (0, ESPAD - ES)),
                            jnp.pad(sdst, (0, ESPAD - ES))]
                           ).reshape(_NW, sgchunks, _CHUNK)
    sdst3d = jnp.pad(sdst, (0, ESPAD - ES), constant_values=_NOUT
                     ).reshape(_NW, sschunks, _CHUNK)
    ssrc3d = jnp.pad(ssrc, (0, ESPAD - ES), constant_values=_NOUT
                     ).reshape(_NW, sschunks, _CHUNK)
    zrow = jnp.zeros((_CHUNK, 128), f32)
    ones128 = jnp.zeros((_CHUNK, 128), f32).at[:, 64].set(1.0)
    pospad = jnp.pad(pos, ((0, 0), (0, 125)))

    xts = [x[:, :, t] for t in range(4)]
    mp = jnp.concatenate([node_mass[:, None], pos], axis=-1)
    grid_n = n // N_BLK
    wf2e, bf2e = _pad_cols(Wf2), _pad_vec(bf2)
    h_topo = pl.pallas_call(
        _node_enc_kernel,
        grid=(grid_n,),
        in_specs=[_row_spec(N_BLK, 12)] * 4 + [_row_spec(N_BLK, 4)]
        + [_full_spec(*w.shape) for w in
           (wx1, wh1, b1, wx2, wh2, b2, wfh, wfmp, bf1, wf2e, bf2e)],
        out_specs=_row_spec(N_BLK, 128),
        out_shape=jax.ShapeDtypeStruct((n, 128), f32),
        interpret=interpret,
    )(*xts, mp, wx1, wh1, b1, wx2, wh2, b2, wfh, wfmp, bf1, wf2e, bf2e)

    ea_pad = jnp.pad(edge_attr, ((0, EPAD - E), (0, 0)))
    mat_id = ea_pad[:, :1].astype(jnp.int32)
    attr = ea_pad[:, 1:]
    edge_feat = _tc_call(
        _edge_enc_kernel, EPAD // E_BLK,
        [(mat_id, E_BLK), (attr, E_BLK)],
        [wemb, wattr, be1, _pad_cols(We2), _pad_vec(be2, one_at=64)],
        E_BLK, 128, interpret)

    pg = _sc_gather(pospad, pidx, nbuf=2)
    ef_s = _tc_call(
        _surf_edge_kernel, ESPAD // E_BLK,
        [(pg[:ESPAD], E_BLK), (pg[ESPAD:], E_BLK)],
        [_pad_rows(wrel), wnrm, bs1, _pad_cols(Ws2), _pad_vec(bs2, one_at=64)],
        E_BLK, 128, interpret)
    aggp = _sc_scatter_surf(ef_s, sdst3d, ssrc3d, zrow, ones128)
    h_final = _tc_call(
        _surf_node_kernel, grid_n,
        [(h_topo, N_BLK), (aggp, N_BLK)],
        [e64, _pad_rows(Wn1), bn1, _pad_cols(Wn2), _pad_vec(bn2)],
        N_BLK, 128, interpret)

    for blk in params['gnn']:
        (Wg1, bg1), (Wg2, bg2) = blk['edge']
        (Wb1, bb1), (Wb2, bb2) = blk['node']
        hg = _sc_gather(h_final, gidx)
        edge_feat = _tc_call(
            _gnn_edge_kernel, EPAD // E_BLK,
            [(edge_feat, E_BLK), (hg[:EPAD], E_BLK), (hg[EPAD:], E_BLK)],
            [_pad_rows(Wg1[:64]), _pad_rows(Wg1[64:128]), _pad_rows(Wg1[128:]),
             bg1, _pad_cols(Wg2), _pad_vec(bg2)],
            E_BLK, 128, interpret)
        aggp = _sc_scatter_add(edge_feat, dst3d, zrow)
        h_final = _tc_call(
            _gnn_node_kernel, grid_n,
            [(h_final, N_BLK), (aggp, N_BLK)],
            [_pad_rows(Wb1[:64]), _pad_rows(Wb1[64:]), bb1,
             _pad_cols(Wb2), _pad_vec(bb2)],
            N_BLK, 128, interpret)

    out = _tc_call(
        _dec_kernel, grid_n,
        [(h_final, N_BLK), (x[:, :, -1], N_BLK), (delta_t[:, None], N_BLK)],
        [_pad_rows(Wd1), bd1, Wd2, bd2], N_BLK, 12, interpret)
    return out


# speedup vs baseline: 1.8156x; 1.1180x over previous
import functools

import jax
import jax.numpy as jnp
from jax.experimental import pallas as pl
from jax.experimental.pallas import tpu as pltpu
from jax.experimental.pallas import tpu_sc as plsc

N_BLK = 1000
E_BLK = 4096

_NW = 32
_CHUNK = 128
_GRP = 5
_NOUT = 10000
_NACC = 10008


def _sc_mesh():
    return plsc.VectorSubcoreMesh(core_axis_name="c", subcore_axis_name="s",
                                  num_cores=2, num_subcores=16)


def _sc_gather(table, idx3d, nbuf=4):
    D = table.shape[1]
    nw, nchunks, ck = idx3d.shape
    ngrp = nchunks // nbuf
    rows_w = nchunks * ck
    M = nw * rows_w

    @functools.partial(
        pl.kernel,
        out_type=jax.ShapeDtypeStruct((M, D), jnp.float32),
        mesh=_sc_mesh(),
        scratch_types=[pltpu.VMEM((nchunks, ck), jnp.int32)]
        + [pltpu.VMEM((ck, D), jnp.float32)] * nbuf
        + [pltpu.SemaphoreType.DMA] * nbuf,
    )
    def k(table_hbm, idx_hbm, out_hbm, idx_v, *bufsem):
        bufs, sems = bufsem[:nbuf], bufsem[nbuf:]
        wid = jax.lax.axis_index("s") * 2 + jax.lax.axis_index("c")
        pltpu.sync_copy(idx_hbm.at[wid], idx_v)
        for b in range(nbuf):
            pltpu.async_copy(table_hbm.at[idx_v.at[b]], bufs[b], sems[b])

        def out_ref(c):
            return out_hbm.at[pl.ds(wid * rows_w + c * ck, ck)]

        def body(g, _):
            for b in range(nbuf):
                c = g * nbuf + b
                pltpu.make_async_copy(table_hbm.at[idx_v.at[c]],
                                      bufs[b], sems[b]).wait()
                pltpu.async_copy(bufs[b], out_ref(c), sems[b])
            for b in range(nbuf):
                c = g * nbuf + b

                @pl.when(c + nbuf < nchunks)
                def _():
                    pltpu.make_async_copy(bufs[b], out_ref(c), sems[b]).wait()
                    pltpu.async_copy(table_hbm.at[idx_v.at[c + nbuf]],
                                     bufs[b], sems[b])
            return 0

        jax.lax.fori_loop(0, ngrp, body, 0)
        for b in range(nbuf):
            pltpu.make_async_copy(bufs[b], out_ref(nchunks - nbuf + b),
                                  sems[b]).wait()

    return k(table, idx3d)


def _zero_acc(zrow_hbm, acc_sh, sid, ck):
    full = _NACC // ck
    rem = _NACC - full * ck
    for j in range(5):
        slot = sid * 5 + j

        @pl.when(slot < full)
        def _():
            pltpu.sync_copy(zrow_hbm, acc_sh.at[pl.ds(slot * ck, ck)])

        @pl.when(slot == full)
        def _():
            pltpu.sync_copy(zrow_hbm.at[pl.ds(0, rem)],
                            acc_sh.at[pl.ds(slot * ck, rem)])


def _sc_scatter_add(vals, idx3d, zrow, nbuf=2):
    D = vals.shape[1]
    nw, nchunks, ck = idx3d.shape
    ngrp = nchunks // nbuf
    rows_w = nchunks * ck

    @functools.partial(
        pl.kernel,
        out_type=jax.ShapeDtypeStruct((2, _NOUT, D), jnp.float32),
        mesh=_sc_mesh(),
        scratch_types=[
            pltpu.VMEM((nchunks, ck), jnp.int32),
            pltpu.VMEM_SHARED((_NACC, D), jnp.float32),
        ]
        + [pltpu.VMEM((ck, D), jnp.float32)] * nbuf
        + [pltpu.SemaphoreType.DMA] * nbuf,
    )
    def k(vals_hbm, idx_hbm, zrow_hbm, out_hbm, idx_v, acc_sh, *bufsem):
        bufs, sems = bufsem[:nbuf], bufsem[nbuf:]
        cid = jax.lax.axis_index("c")
        sid = jax.lax.axis_index("s")
        wid = sid * 2 + cid

        _zero_acc(zrow_hbm, acc_sh, sid, ck)
        pltpu.sync_copy(idx_hbm.at[wid], idx_v)
        plsc.subcore_barrier()

        def val_ref(c):
            return vals_hbm.at[pl.ds(wid * rows_w + c * ck, ck)]

        for b in range(nbuf):
            pltpu.async_copy(val_ref(b), bufs[b], sems[b])

        def body(g, _):
            for b in range(nbuf):
                c = g * nbuf + b
                pltpu.make_async_copy(val_ref(c), bufs[b], sems[b]).wait()
                pltpu.async_copy(bufs[b], acc_sh.at[idx_v.at[c]], sems[b],
                                 add=True)
            for b in range(nbuf):
                c = g * nbuf + b

                @pl.when(c + nbuf < nchunks)
                def _():
                    pltpu.make_async_copy(bufs[b], acc_sh.at[idx_v.at[c]],
                                          sems[b]).wait()
                    pltpu.async_copy(val_ref(c + nbuf), bufs[b], sems[b])
            return 0

        jax.lax.fori_loop(0, ngrp, body, 0)
        for b in range(nbuf):
            pltpu.make_async_copy(bufs[b],
                                  acc_sh.at[idx_v.at[nchunks - nbuf + b]],
                                  sems[b]).wait()
        plsc.subcore_barrier()

        @pl.when(sid == 0)
        def _():
            pltpu.sync_copy(acc_sh.at[pl.ds(0, _NOUT)], out_hbm.at[cid])

    return k(vals, idx3d, zrow)


def _sc_scatter_surf(vals, dst3d, src3d, zrow, ones_hbm):
    D = vals.shape[1]
    nw, nchunks, ck = dst3d.shape
    rows_w = nchunks * ck

    @functools.partial(
        pl.kernel,
        out_type=jax.ShapeDtypeStruct((2, _NOUT, D), jnp.float32),
        mesh=_sc_mesh(),
        scratch_types=[
            pltpu.VMEM((nchunks, ck), jnp.int32),
            pltpu.VMEM((nchunks, ck), jnp.int32),
            pltpu.VMEM((ck, D), jnp.float32),
            pltpu.VMEM((ck, D), jnp.float32),
            pltpu.VMEM_SHARED((_NACC, D), jnp.float32),
            pltpu.SemaphoreType.DMA,
        ],
    )
    def k(vals_hbm, dst_hbm, src_hbm, zrow_hbm, ones_h,
          out_hbm, idxd_v, idxs_v, rows_v, ones_v, acc_sh, sem):
        cid = jax.lax.axis_index("c")
        sid = jax.lax.axis_index("s")
        wid = sid * 2 + cid

        _zero_acc(zrow_hbm, acc_sh, sid, ck)
        pltpu.sync_copy(dst_hbm.at[wid], idxd_v)
        pltpu.sync_copy(src_hbm.at[wid], idxs_v)
        pltpu.sync_copy(ones_h, ones_v)
        plsc.subcore_barrier()
        for i in range(nchunks):
            pltpu.sync_copy(vals_hbm.at[pl.ds(wid * rows_w + i * ck, ck)], rows_v)
            pltpu.sync_copy(rows_v, acc_sh.at[idxd_v.at[i]], add=True)
            pltpu.sync_copy(ones_v, acc_sh.at[idxs_v.at[i]], add=True)
        plsc.subcore_barrier()

        @pl.when(sid == 0)
        def _():
            pltpu.sync_copy(acc_sh.at[pl.ds(0, _NOUT)], out_hbm.at[cid])

    return k(vals, dst3d, src3d, zrow, ones_hbm)


def _node_enc_kernel(x0, x1, x2, x3, mp,
                     wx1, wh1, b1, wx2, wh2, b2,
                     wfh, wfmp, bf1, wf2, bf2, o_ref):
    xs = (x0[...], x1[...], x2[...], x3[...])
    B = xs[0].shape[0]
    dt = jnp.float32

    def lstm_layer(inputs, wx, wh, b):
        h = jnp.zeros((B, 64), dt)
        c = jnp.zeros((B, 64), dt)
        outs = []
        for xt in inputs:
            g = xt @ wx[...] + h @ wh[...] + b[...]
            i, f, gg, o = jnp.split(g, 4, axis=-1)
            c = jax.nn.sigmoid(f) * c + jax.nn.sigmoid(i) * jnp.tanh(gg)
            h = jax.nn.sigmoid(o) * jnp.tanh(c)
            outs.append(h)
        return outs

    h1 = lstm_layer(xs, wx1, wh1, b1)
    h2 = lstm_layer(h1, wx2, wh2, b2)
    h = h2[-1]
    a = jnp.maximum(h @ wfh[...] + mp[...] @ wfmp[...] + bf1[...], 0.0)
    o_ref[...] = a @ wf2[...] + bf2[...]


def _edge_enc_kernel(mid, attr, wemb, wattr, b1, w2, b2, o_ref):
    B = mid.shape[0]
    oh = (mid[...] == jax.lax.broadcasted_iota(jnp.int32, (B, 8), 1)).astype(jnp.float32)
    a = jnp.maximum(oh @ wemb[...] + attr[...] @ wattr[...] + b1[...], 0.0)
    o_ref[...] = a @ w2[...] + b2[...]


def _surf_edge_kernel(psrc, pdst, wrel, wnrm, b1, w2, b2, o_ref):
    rel = psrc[...] - pdst[...]
    nrm = jnp.sqrt(jnp.sum(rel * rel, axis=-1, keepdims=True) + 1e-12)
    a = jnp.maximum(rel @ wrel[...] + nrm @ wnrm[...] + b1[...], 0.0)
    o_ref[...] = a @ w2[...] + b2[...]


def _surf_node_kernel(htopo, aggp, e64, w1, b1, w2, b2, o_ref):
    u = aggp[0] + aggp[1]
    a = jnp.maximum(u @ w1[...] + b1[...], 0.0)
    hs = a @ w2[...] + b2[...]
    mask = (u @ e64[...] > 0.0).astype(jnp.float32)
    o_ref[...] = htopo[...] + hs * mask


def _gnn_edge_kernel(ef, hs, hd, w0e, w0s, w0d, b0, w2, b2, o_ref):
    a = jnp.maximum(ef[...] @ w0e[...] + hs[...] @ w0s[...] + hd[...] @ w0d[...]
                    + b0[...], 0.0)
    o_ref[...] = ef[...] + a @ w2[...] + b2[...]


def _gnn_node_kernel(h, aggpa, aggpb, w1h, w1a, b1, w2, b2, o_ref):
    agg = aggpa[0] + aggpa[1] + aggpb[0] + aggpb[1]
    a = jnp.maximum(h[...] @ w1h[...] + agg @ w1a[...] + b1[...], 0.0)
    o_ref[...] = h[...] + a @ w2[...] + b2[...]


def _dec_kernel(h, xt, dt, w1, b1, w2, b2, o_ref):
    a = jnp.maximum(h[...] @ w1[...] + b1[...], 0.0)
    d = a @ w2[...] + b2[...]
    o_ref[...] = xt[...] + d * dt[...]


def _row_spec(b, *rest):
    nd = 1 + len(rest)
    if nd == 2:
        return pl.BlockSpec((b, rest[0]), lambda i: (i, 0))
    return pl.BlockSpec((rest[0], b, rest[1]), lambda i: (0, i, 0))


def _row_spec_off(b, cols, off):
    return pl.BlockSpec((b, cols), lambda i, off=off: (i + off, 0))


def _full_spec(*shape):
    nd = len(shape)
    return pl.BlockSpec(shape, lambda i: (0,) * nd)


def _tc_call(kern, grid, row_args, full_args, out_rows, out_cols,
             interpret=False):
    in_specs = []
    for a, b in row_args:
        if a.ndim == 2:
            in_specs.append(_row_spec(b, a.shape[-1]))
        else:
            in_specs.append(_row_spec(b, a.shape[0], a.shape[-1]))
    in_specs += [_full_spec(*a.shape) for a in full_args]
    return pl.pallas_call(
        kern,
        grid=(grid,),
        in_specs=in_specs,
        out_specs=_row_spec(out_rows, out_cols),
        out_shape=jax.ShapeDtypeStruct((grid * out_rows, out_cols), jnp.float32),
        interpret=interpret,
    )(*[a for a, _ in row_args], *full_args)


def _pad_rows(w, rows=128):
    return jnp.pad(w, ((0, rows - w.shape[0]), (0, 0)))


def _pad_cols(w, cols=128):
    return jnp.pad(w, ((0, 0), (0, cols - w.shape[1])))


def _pad_vec(b, cols=128, one_at=None):
    v = jnp.pad(b, (0, cols - b.shape[0]))
    if one_at is not None:
        v = v.at[one_at].set(1.0)
    return v


def kernel(x, node_mass, pos, edge_attr, delta_t, params, edge_index,
           edge_surf_index, interpret=False):
    n = x.shape[0]
    E = edge_index.shape[1]
    ES = edge_surf_index.shape[1]
    f32 = jnp.float32
    EPAD = ((E + _NW * _CHUNK - 1) // (_NW * _CHUNK)) * _NW * _CHUNK
    ESPAD = ((ES + _NW * _CHUNK - 1) // (_NW * _CHUNK)) * _NW * _CHUNK

    (Wih1, Whh1, bih1, bhh1), (Wih2, Whh2, bih2, bhh2) = params['lstm']
    wx1, wh1, b1 = Wih1.T, Whh1.T, bih1 + bhh1
    wx2, wh2, b2 = Wih2.T, Whh2.T, bih2 + bhh2
    (Wf1, bf1), (Wf2, bf2) = params['temp_fc']
    wfh, wfmp = Wf1[:64], Wf1[64:]
    (We1, be1), (We2, be2) = params['edge_mlp']
    wemb = params['mat_emb'] @ We1[:4]
    wattr = We1[4:]
    (Ws1, bs1), (Ws2, bs2) = params['surf_edge']
    wrel, wnrm = Ws1[:3], Ws1[3:]
    (Wn1, bn1), (Wn2, bn2) = params['surf_node']
    (Wd1, bd1), (Wd2, bd2) = params['dec']
    e64 = jnp.zeros((128, 1), f32).at[64, 0].set(1.0)

    src = edge_index[0].astype(jnp.int32)
    dst = edge_index[1].astype(jnp.int32)
    ssrc = edge_surf_index[0].astype(jnp.int32)
    sdst = edge_surf_index[1].astype(jnp.int32)
    EPAD2 = EPAD // 2
    ghalf = 2 * EPAD2 // (_NW * _CHUNK)
    shalf = EPAD2 // (_NW * _CHUNK)
    sgchunks = 2 * ESPAD // (_NW * _CHUNK)
    sschunks = ESPAD // (_NW * _CHUNK)
    srcp = jnp.pad(src, (0, EPAD - E))
    dstp = jnp.pad(dst, (0, EPAD - E))
    dstpt = jnp.pad(dst, (0, EPAD - E), constant_values=_NOUT)
    gidxh = [jnp.concatenate([srcp[h * EPAD2:(h + 1) * EPAD2],
                              dstp[h * EPAD2:(h + 1) * EPAD2]]
                             ).reshape(_NW, ghalf, _CHUNK) for h in range(2)]
    dst3dh = [dstpt[h * EPAD2:(h + 1) * EPAD2].reshape(_NW, shalf, _CHUNK)
              for h in range(2)]
    pidx = jnp.concatenate([jnp.pad(ssrc, (0, ESPAD - ES)),
                            jnp.pad(sdst, (0, ESPAD - ES))]
                           ).reshape(_NW, sgchunks, _CHUNK)
    sdst3d = jnp.pad(sdst, (0, ESPAD - ES), constant_values=_NOUT
                     ).reshape(_NW, sschunks, _CHUNK)
    ssrc3d = jnp.pad(ssrc, (0, ESPAD - ES), constant_values=_NOUT
                     ).reshape(_NW, sschunks, _CHUNK)
    zrow = jnp.zeros((_CHUNK, 128), f32)
    ones128 = jnp.zeros((_CHUNK, 128), f32).at[:, 64].set(1.0)
    pospad = jnp.pad(pos, ((0, 0), (0, 125)))

    xts = [x[:, :, t] for t in range(4)]
    mp = jnp.concatenate([node_mass[:, None], pos], axis=-1)
    grid_n = n // N_BLK
    wf2e, bf2e = _pad_cols(Wf2), _pad_vec(bf2)
    h_topo = pl.pallas_call(
        _node_enc_kernel,
        grid=(grid_n,),
        in_specs=[_row_spec(N_BLK, 12)] * 4 + [_row_spec(N_BLK, 4)]
        + [_full_spec(*w.shape) for w in
           (wx1, wh1, b1, wx2, wh2, b2, wfh, wfmp, bf1, wf2e, bf2e)],
        out_specs=_row_spec(N_BLK, 128),
        out_shape=jax.ShapeDtypeStruct((n, 128), f32),
        interpret=interpret,
    )(*xts, mp, wx1, wh1, b1, wx2, wh2, b2, wfh, wfmp, bf1, wf2e, bf2e)

    ea_pad = jnp.pad(edge_attr, ((0, EPAD - E), (0, 0)))
    mat_id = ea_pad[:, :1].astype(jnp.int32)
    attr = ea_pad[:, 1:]
    edge_feat = _tc_call(
        _edge_enc_kernel, EPAD // E_BLK,
        [(mat_id, E_BLK), (attr, E_BLK)],
        [wemb, wattr, be1, _pad_cols(We2), _pad_vec(be2, one_at=64)],
        E_BLK, 128, interpret)

    pg = _sc_gather(pospad, pidx, nbuf=2)
    surf_w = [_pad_rows(wrel), wnrm, bs1, _pad_cols(Ws2),
              _pad_vec(bs2, one_at=64)]
    nbs = ESPAD // E_BLK
    ef_s = pl.pallas_call(
        _surf_edge_kernel,
        grid=(nbs,),
        in_specs=[_row_spec_off(E_BLK, 128, 0), _row_spec_off(E_BLK, 128, nbs)]
        + [_full_spec(*w.shape) for w in surf_w],
        out_specs=_row_spec(E_BLK, 128),
        out_shape=jax.ShapeDtypeStruct((ESPAD, 128), f32),
        interpret=interpret,
    )(pg, pg, *surf_w)
    aggp = _sc_scatter_surf(ef_s, sdst3d, ssrc3d, zrow, ones128)
    h_final = _tc_call(
        _surf_node_kernel, grid_n,
        [(h_topo, N_BLK), (aggp, N_BLK)],
        [e64, _pad_rows(Wn1), bn1, _pad_cols(Wn2), _pad_vec(bn2)],
        N_BLK, 128, interpret)

    nbh = EPAD2 // E_BLK
    ef_parts = [(edge_feat, 0), (edge_feat, nbh)]
    for blk in params['gnn']:
        (Wg1, bg1), (Wg2, bg2) = blk['edge']
        (Wb1, bb1), (Wb2, bb2) = blk['node']
        wgs = [_pad_rows(Wg1[:64]), _pad_rows(Wg1[64:128]),
               _pad_rows(Wg1[128:]), bg1, _pad_cols(Wg2), _pad_vec(bg2)]
        hg = [_sc_gather(h_final, gidxh[0]), _sc_gather(h_final, gidxh[1])]
        new_parts = []
        for h in range(2):
            efa, off = ef_parts[h]
            out = pl.pallas_call(
                _gnn_edge_kernel,
                grid=(nbh,),
                in_specs=[_row_spec_off(E_BLK, 128, off),
                          _row_spec_off(E_BLK, 128, 0),
                          _row_spec_off(E_BLK, 128, nbh)]
                + [_full_spec(*w.shape) for w in wgs],
                out_specs=_row_spec(E_BLK, 128),
                out_shape=jax.ShapeDtypeStruct((EPAD2, 128), f32),
                interpret=interpret,
            )(efa, hg[h], hg[h], *wgs)
            new_parts.append((out, 0))
        ef_parts = new_parts
        aggs = [_sc_scatter_add(ef_parts[h][0], dst3dh[h], zrow)
                for h in range(2)]
        h_final = _tc_call(
            _gnn_node_kernel, grid_n,
            [(h_final, N_BLK), (aggs[0], N_BLK), (aggs[1], N_BLK)],
            [_pad_rows(Wb1[:64]), _pad_rows(Wb1[64:]), bb1,
             _pad_cols(Wb2), _pad_vec(bb2)],
            N_BLK, 128, interpret)

    out = _tc_call(
        _dec_kernel, grid_n,
        [(h_final, N_BLK), (x[:, :, -1], N_BLK), (delta_t[:, None], N_BLK)],
        [_pad_rows(Wd1), bd1, Wd2, bd2], N_BLK, 12, interpret)
    return out


# speedup vs baseline: 1.8166x; 1.0005x over previous
import functools

import jax
import jax.numpy as jnp
from jax.experimental import pallas as pl
from jax.experimental.pallas import tpu as pltpu
from jax.experimental.pallas import tpu_sc as plsc

N_BLK = 1000
E_BLK = 4096

_NW = 32
_CHUNK = 128
_GRP = 5
_NOUT = 10000
_NACC = 10008


def _sc_mesh():
    return plsc.VectorSubcoreMesh(core_axis_name="c", subcore_axis_name="s",
                                  num_cores=2, num_subcores=16)


def _sc_gather(table, idx3d, nbuf=4):
    D = table.shape[1]
    nw, nchunks, ck = idx3d.shape
    ngrp = nchunks // nbuf
    rows_w = nchunks * ck
    M = nw * rows_w

    @functools.partial(
        pl.kernel,
        out_type=jax.ShapeDtypeStruct((M, D), jnp.float32),
        mesh=_sc_mesh(),
        scratch_types=[pltpu.VMEM((nchunks, ck), jnp.int32)]
        + [pltpu.VMEM((ck, D), jnp.float32)] * nbuf
        + [pltpu.SemaphoreType.DMA] * nbuf,
    )
    def k(table_hbm, idx_hbm, out_hbm, idx_v, *bufsem):
        bufs, sems = bufsem[:nbuf], bufsem[nbuf:]
        wid = jax.lax.axis_index("s") * 2 + jax.lax.axis_index("c")
        pltpu.sync_copy(idx_hbm.at[wid], idx_v)
        for b in range(nbuf):
            pltpu.async_copy(table_hbm.at[idx_v.at[b]], bufs[b], sems[b])

        def out_ref(c):
            return out_hbm.at[pl.ds(wid * rows_w + c * ck, ck)]

        def body(g, _):
            for b in range(nbuf):
                c = g * nbuf + b
                pltpu.make_async_copy(table_hbm.at[idx_v.at[c]],
                                      bufs[b], sems[b]).wait()
                pltpu.async_copy(bufs[b], out_ref(c), sems[b])
            for b in range(nbuf):
                c = g * nbuf + b

                @pl.when(c + nbuf < nchunks)
                def _():
                    pltpu.make_async_copy(bufs[b], out_ref(c), sems[b]).wait()
                    pltpu.async_copy(table_hbm.at[idx_v.at[c + nbuf]],
                                     bufs[b], sems[b])
            return 0

        jax.lax.fori_loop(0, ngrp, body, 0)
        for b in range(nbuf):
            pltpu.make_async_copy(bufs[b], out_ref(nchunks - nbuf + b),
                                  sems[b]).wait()

    return k(table, idx3d)


def _zero_acc(zrow_hbm, acc_sh, sid, ck):
    full = _NACC // ck
    rem = _NACC - full * ck
    for j in range(5):
        slot = sid * 5 + j

        @pl.when(slot < full)
        def _():
            pltpu.sync_copy(zrow_hbm, acc_sh.at[pl.ds(slot * ck, ck)])

        @pl.when(slot == full)
        def _():
            pltpu.sync_copy(zrow_hbm.at[pl.ds(0, rem)],
                            acc_sh.at[pl.ds(slot * ck, rem)])


def _copy_acc_out(acc_sh, out_hbm, cid, sid):
    @pl.when(sid < 15)
    def _():
        pltpu.sync_copy(acc_sh.at[pl.ds(sid * 624, 624)],
                        out_hbm.at[cid, pl.ds(sid * 624, 624)])

    @pl.when(sid == 15)
    def _():
        pltpu.sync_copy(acc_sh.at[pl.ds(9360, _NOUT - 9360)],
                        out_hbm.at[cid, pl.ds(9360, _NOUT - 9360)])


def _sc_scatter_add(vals, idx3d, zrow, nbuf=2):
    D = vals.shape[1]
    nw, nchunks, ck = idx3d.shape
    ngrp = nchunks // nbuf
    rows_w = nchunks * ck

    @functools.partial(
        pl.kernel,
        out_type=jax.ShapeDtypeStruct((2, _NOUT, D), jnp.float32),
        mesh=_sc_mesh(),
        scratch_types=[
            pltpu.VMEM((nchunks, ck), jnp.int32),
            pltpu.VMEM_SHARED((_NACC, D), jnp.float32),
        ]
        + [pltpu.VMEM((ck, D), jnp.float32)] * nbuf
        + [pltpu.SemaphoreType.DMA] * nbuf,
    )
    def k(vals_hbm, idx_hbm, zrow_hbm, out_hbm, idx_v, acc_sh, *bufsem):
        bufs, sems = bufsem[:nbuf], bufsem[nbuf:]
        cid = jax.lax.axis_index("c")
        sid = jax.lax.axis_index("s")
        wid = sid * 2 + cid

        _zero_acc(zrow_hbm, acc_sh, sid, ck)
        pltpu.sync_copy(idx_hbm.at[wid], idx_v)
        plsc.subcore_barrier()

        def val_ref(c):
            return vals_hbm.at[pl.ds(wid * rows_w + c * ck, ck)]

        for b in range(nbuf):
            pltpu.async_copy(val_ref(b), bufs[b], sems[b])

        def body(g, _):
            for b in range(nbuf):
                c = g * nbuf + b
                pltpu.make_async_copy(val_ref(c), bufs[b], sems[b]).wait()
                pltpu.async_copy(bufs[b], acc_sh.at[idx_v.at[c]], sems[b],
                                 add=True)
            for b in range(nbuf):
                c = g * nbuf + b

                @pl.when(c + nbuf < nchunks)
                def _():
                    pltpu.make_async_copy(bufs[b], acc_sh.at[idx_v.at[c]],
                                          sems[b]).wait()
                    pltpu.async_copy(val_ref(c + nbuf), bufs[b], sems[b])
            return 0

        jax.lax.fori_loop(0, ngrp, body, 0)
        for b in range(nbuf):
            pltpu.make_async_copy(bufs[b],
                                  acc_sh.at[idx_v.at[nchunks - nbuf + b]],
                                  sems[b]).wait()
        plsc.subcore_barrier()
        _copy_acc_out(acc_sh, out_hbm, cid, sid)

    return k(vals, idx3d, zrow)


def _sc_scatter_surf(vals, dst3d, src3d, zrow, ones_hbm):
    D = vals.shape[1]
    nw, nchunks, ck = dst3d.shape
    rows_w = nchunks * ck

    @functools.partial(
        pl.kernel,
        out_type=jax.ShapeDtypeStruct((2, _NOUT, D), jnp.float32),
        mesh=_sc_mesh(),
        scratch_types=[
            pltpu.VMEM((nchunks, ck), jnp.int32),
            pltpu.VMEM((nchunks, ck), jnp.int32),
            pltpu.VMEM((ck, D), jnp.float32),
            pltpu.VMEM((ck, D), jnp.float32),
            pltpu.VMEM_SHARED((_NACC, D), jnp.float32),
            pltpu.SemaphoreType.DMA,
        ],
    )
    def k(vals_hbm, dst_hbm, src_hbm, zrow_hbm, ones_h,
          out_hbm, idxd_v, idxs_v, rows_v, ones_v, acc_sh, sem):
        cid = jax.lax.axis_index("c")
        sid = jax.lax.axis_index("s")
        wid = sid * 2 + cid

        _zero_acc(zrow_hbm, acc_sh, sid, ck)
        pltpu.sync_copy(dst_hbm.at[wid], idxd_v)
        pltpu.sync_copy(src_hbm.at[wid], idxs_v)
        pltpu.sync_copy(ones_h, ones_v)
        plsc.subcore_barrier()
        for i in range(nchunks):
            pltpu.sync_copy(vals_hbm.at[pl.ds(wid * rows_w + i * ck, ck)], rows_v)
            pltpu.sync_copy(rows_v, acc_sh.at[idxd_v.at[i]], add=True)
            pltpu.sync_copy(ones_v, acc_sh.at[idxs_v.at[i]], add=True)
        plsc.subcore_barrier()
        _copy_acc_out(acc_sh, out_hbm, cid, sid)

    return k(vals, dst3d, src3d, zrow, ones_hbm)


def _node_enc_kernel(x0, x1, x2, x3, mp,
                     wx1, wh1, b1, wx2, wh2, b2,
                     wfh, wfmp, bf1, wf2, bf2, o_ref):
    xs = (x0[...], x1[...], x2[...], x3[...])
    B = xs[0].shape[0]
    dt = jnp.float32

    def lstm_layer(inputs, wx, wh, b):
        h = jnp.zeros((B, 64), dt)
        c = jnp.zeros((B, 64), dt)
        outs = []
        for xt in inputs:
            g = xt @ wx[...] + h @ wh[...] + b[...]
            i, f, gg, o = jnp.split(g, 4, axis=-1)
            c = jax.nn.sigmoid(f) * c + jax.nn.sigmoid(i) * jnp.tanh(gg)
            h = jax.nn.sigmoid(o) * jnp.tanh(c)
            outs.append(h)
        return outs

    h1 = lstm_layer(xs, wx1, wh1, b1)
    h2 = lstm_layer(h1, wx2, wh2, b2)
    h = h2[-1]
    a = jnp.maximum(h @ wfh[...] + mp[...] @ wfmp[...] + bf1[...], 0.0)
    o_ref[...] = a @ wf2[...] + bf2[...]


def _edge_enc_kernel(mid, attr, wemb, wattr, b1, w2, b2, o_ref):
    B = mid.shape[0]
    oh = (mid[...] == jax.lax.broadcasted_iota(jnp.int32, (B, 8), 1)).astype(jnp.float32)
    a = jnp.maximum(oh @ wemb[...] + attr[...] @ wattr[...] + b1[...], 0.0)
    o_ref[...] = a @ w2[...] + b2[...]


def _surf_edge_kernel(psrc, pdst, wrel, wnrm, b1, w2, b2, o_ref):
    rel = psrc[...] - pdst[...]
    nrm = jnp.sqrt(jnp.sum(rel * rel, axis=-1, keepdims=True) + 1e-12)
    a = jnp.maximum(rel @ wrel[...] + nrm @ wnrm[...] + b1[...], 0.0)
    o_ref[...] = a @ w2[...] + b2[...]


def _surf_node_kernel(htopo, aggp, e64, w1, b1, w2, b2, o_ref):
    u = aggp[0] + aggp[1]
    a = jnp.maximum(u @ w1[...] + b1[...], 0.0)
    hs = a @ w2[...] + b2[...]
    mask = (u @ e64[...] > 0.0).astype(jnp.float32)
    o_ref[...] = htopo[...] + hs * mask


def _gnn_edge_kernel(ef, hs, hd, w0e, w0s, w0d, b0, w2, b2, o_ref):
    a = jnp.maximum(ef[...] @ w0e[...] + hs[...] @ w0s[...] + hd[...] @ w0d[...]
                    + b0[...], 0.0)
    o_ref[...] = ef[...] + a @ w2[...] + b2[...]


def _gnn_node_kernel(h, aggpa, aggpb, w1h, w1a, b1, w2, b2, o_ref):
    agg = aggpa[0] + aggpa[1] + aggpb[0] + aggpb[1]
    a = jnp.maximum(h[...] @ w1h[...] + agg @ w1a[...] + b1[...], 0.0)
    o_ref[...] = h[...] + a @ w2[...] + b2[...]


def _dec_kernel(h, xt, dt, w1, b1, w2, b2, o_ref):
    a = jnp.maximum(h[...] @ w1[...] + b1[...], 0.0)
    d = a @ w2[...] + b2[...]
    o_ref[...] = xt[...] + d * dt[...]


def _row_spec(b, *rest):
    nd = 1 + len(rest)
    if nd == 2:
        return pl.BlockSpec((b, rest[0]), lambda i: (i, 0))
    return pl.BlockSpec((rest[0], b, rest[1]), lambda i: (0, i, 0))


def _row_spec_off(b, cols, off):
    return pl.BlockSpec((b, cols), lambda i, off=off: (i + off, 0))


def _full_spec(*shape):
    nd = len(shape)
    return pl.BlockSpec(shape, lambda i: (0,) * nd)


def _tc_call(kern, grid, row_args, full_args, out_rows, out_cols,
             interpret=False):
    in_specs = []
    for a, b in row_args:
        if a.ndim == 2:
            in_specs.append(_row_spec(b, a.shape[-1]))
        else:
            in_specs.append(_row_spec(b, a.shape[0], a.shape[-1]))
    in_specs += [_full_spec(*a.shape) for a in full_args]
    return pl.pallas_call(
        kern,
        grid=(grid,),
        in_specs=in_specs,
        out_specs=_row_spec(out_rows, out_cols),
        out_shape=jax.ShapeDtypeStruct((grid * out_rows, out_cols), jnp.float32),
        interpret=interpret,
    )(*[a for a, _ in row_args], *full_args)


def _pad_rows(w, rows=128):
    return jnp.pad(w, ((0, rows - w.shape[0]), (0, 0)))


def _pad_cols(w, cols=128):
    return jnp.pad(w, ((0, 0), (0, cols - w.shape[1])))


def _pad_vec(b, cols=128, one_at=None):
    v = jnp.pad(b, (0, cols - b.shape[0]))
    if one_at is not None:
        v = v.at[one_at].set(1.0)
    return v


def kernel(x, node_mass, pos, edge_attr, delta_t, params, edge_index,
           edge_surf_index, interpret=False):
    n = x.shape[0]
    E = edge_index.shape[1]
    ES = edge_surf_index.shape[1]
    f32 = jnp.float32
    EPAD = ((E + _NW * _CHUNK - 1) // (_NW * _CHUNK)) * _NW * _CHUNK
    ESPAD = ((ES + _NW * _CHUNK - 1) // (_NW * _CHUNK)) * _NW * _CHUNK

    (Wih1, Whh1, bih1, bhh1), (Wih2, Whh2, bih2, bhh2) = params['lstm']
    wx1, wh1, b1 = Wih1.T, Whh1.T, bih1 + bhh1
    wx2, wh2, b2 = Wih2.T, Whh2.T, bih2 + bhh2
    (Wf1, bf1), (Wf2, bf2) = params['temp_fc']
    wfh, wfmp = Wf1[:64], Wf1[64:]
    (We1, be1), (We2, be2) = params['edge_mlp']
    wemb = params['mat_emb'] @ We1[:4]
    wattr = We1[4:]
    (Ws1, bs1), (Ws2, bs2) = params['surf_edge']
    wrel, wnrm = Ws1[:3], Ws1[3:]
    (Wn1, bn1), (Wn2, bn2) = params['surf_node']
    (Wd1, bd1), (Wd2, bd2) = params['dec']
    e64 = jnp.zeros((128, 1), f32).at[64, 0].set(1.0)

    src = edge_index[0].astype(jnp.int32)
    dst = edge_index[1].astype(jnp.int32)
    ssrc = edge_surf_index[0].astype(jnp.int32)
    sdst = edge_surf_index[1].astype(jnp.int32)
    EPAD2 = EPAD // 2
    ghalf = 2 * EPAD2 // (_NW * _CHUNK)
    shalf = EPAD2 // (_NW * _CHUNK)
    sgchunks = 2 * ESPAD // (_NW * _CHUNK)
    sschunks = ESPAD // (_NW * _CHUNK)
    srcp = jnp.pad(src, (0, EPAD - E))
    dstp = jnp.pad(dst, (0, EPAD - E))
    dstpt = jnp.pad(dst, (0, EPAD - E), constant_values=_NOUT)
    gidxh = [jnp.concatenate([srcp[h * EPAD2:(h + 1) * EPAD2],
                              dstp[h * EPAD2:(h + 1) * EPAD2]]
                             ).reshape(_NW, ghalf, _CHUNK) for h in range(2)]
    dst3dh = [dstpt[h * EPAD2:(h + 1) * EPAD2].reshape(_NW, shalf, _CHUNK)
              for h in range(2)]
    pidx = jnp.concatenate([jnp.pad(ssrc, (0, ESPAD - ES)),
                            jnp.pad(sdst, (0, ESPAD - ES))]
                           ).reshape(_NW, sgchunks, _CHUNK)
    sdst3d = jnp.pad(sdst, (0, ESPAD - ES), constant_values=_NOUT
                     ).reshape(_NW, sschunks, _CHUNK)
    ssrc3d = jnp.pad(ssrc, (0, ESPAD - ES), constant_values=_NOUT
                     ).reshape(_NW, sschunks, _CHUNK)
    zrow = jnp.zeros((_CHUNK, 128), f32)
    ones128 = jnp.zeros((_CHUNK, 128), f32).at[:, 64].set(1.0)
    pospad = jnp.pad(pos, ((0, 0), (0, 125)))

    xts = [x[:, :, t] for t in range(4)]
    mp = jnp.concatenate([node_mass[:, None], pos], axis=-1)
    grid_n = n // N_BLK
    wf2e, bf2e = _pad_cols(Wf2), _pad_vec(bf2)
    h_topo = pl.pallas_call(
        _node_enc_kernel,
        grid=(grid_n,),
        in_specs=[_row_spec(N_BLK, 12)] * 4 + [_row_spec(N_BLK, 4)]
        + [_full_spec(*w.shape) for w in
           (wx1, wh1, b1, wx2, wh2, b2, wfh, wfmp, bf1, wf2e, bf2e)],
        out_specs=_row_spec(N_BLK, 128),
        out_shape=jax.ShapeDtypeStruct((n, 128), f32),
        interpret=interpret,
    )(*xts, mp, wx1, wh1, b1, wx2, wh2, b2, wfh, wfmp, bf1, wf2e, bf2e)

    ea_pad = jnp.pad(edge_attr, ((0, EPAD - E), (0, 0)))
    mat_id = ea_pad[:, :1].astype(jnp.int32)
    attr = ea_pad[:, 1:]
    edge_feat = _tc_call(
        _edge_enc_kernel, EPAD // E_BLK,
        [(mat_id, E_BLK), (attr, E_BLK)],
        [wemb, wattr, be1, _pad_cols(We2), _pad_vec(be2, one_at=64)],
        E_BLK, 128, interpret)

    pg = _sc_gather(pospad, pidx, nbuf=2)
    surf_w = [_pad_rows(wrel), wnrm, bs1, _pad_cols(Ws2),
              _pad_vec(bs2, one_at=64)]
    nbs = ESPAD // E_BLK
    ef_s = pl.pallas_call(
        _surf_edge_kernel,
        grid=(nbs,),
        in_specs=[_row_spec_off(E_BLK, 128, 0), _row_spec_off(E_BLK, 128, nbs)]
        + [_full_spec(*w.shape) for w in surf_w],
        out_specs=_row_spec(E_BLK, 128),
        out_shape=jax.ShapeDtypeStruct((ESPAD, 128), f32),
        interpret=interpret,
    )(pg, pg, *surf_w)
    aggp = _sc_scatter_surf(ef_s, sdst3d, ssrc3d, zrow, ones128)
    h_final = _tc_call(
        _surf_node_kernel, grid_n,
        [(h_topo, N_BLK), (aggp, N_BLK)],
        [e64, _pad_rows(Wn1), bn1, _pad_cols(Wn2), _pad_vec(bn2)],
        N_BLK, 128, interpret)

    nbh = EPAD2 // E_BLK
    ef_parts = [(edge_feat, 0), (edge_feat, nbh)]
    for blk in params['gnn']:
        (Wg1, bg1), (Wg2, bg2) = blk['edge']
        (Wb1, bb1), (Wb2, bb2) = blk['node']
        wgs = [_pad_rows(Wg1[:64]), _pad_rows(Wg1[64:128]),
               _pad_rows(Wg1[128:]), bg1, _pad_cols(Wg2), _pad_vec(bg2)]
        hg = [_sc_gather(h_final, gidxh[0]), _sc_gather(h_final, gidxh[1])]
        new_parts = []
        for h in range(2):
            efa, off = ef_parts[h]
            out = pl.pallas_call(
                _gnn_edge_kernel,
                grid=(nbh,),
                in_specs=[_row_spec_off(E_BLK, 128, off),
                          _row_spec_off(E_BLK, 128, 0),
                          _row_spec_off(E_BLK, 128, nbh)]
                + [_full_spec(*w.shape) for w in wgs],
                out_specs=_row_spec(E_BLK, 128),
                out_shape=jax.ShapeDtypeStruct((EPAD2, 128), f32),
                interpret=interpret,
            )(efa, hg[h], hg[h], *wgs)
            new_parts.append((out, 0))
        ef_parts = new_parts
        aggs = [_sc_scatter_add(ef_parts[h][0], dst3dh[h], zrow)
                for h in range(2)]
        h_final = _tc_call(
            _gnn_node_kernel, grid_n,
            [(h_final, N_BLK), (aggs[0], N_BLK), (aggs[1], N_BLK)],
            [_pad_rows(Wb1[:64]), _pad_rows(Wb1[64:]), bb1,
             _pad_cols(Wb2), _pad_vec(bb2)],
            N_BLK, 128, interpret)

    out = _tc_call(
        _dec_kernel, grid_n,
        [(h_final, N_BLK), (x[:, :, -1], N_BLK), (delta_t[:, None], N_BLK)],
        [_pad_rows(Wd1), bd1, Wd2, bd2], N_BLK, 12, interpret)
    return out
